# Initial kernel scaffold; baseline (speedup 1.0000x reference)
import jax
import jax.numpy as jnp
from jax.experimental import pallas as pl


def kernel(feats_a, feats_b, feats_idx, s_inv):
    raise NotImplementedError("write your pallas kernel here")


# speedup vs baseline: 1.6265x; 1.6265x over previous
import functools

import jax
import jax.numpy as jnp
from jax import lax
from jax.experimental import pallas as pl
from jax.experimental.pallas import tpu as pltpu
from jax.experimental.pallas import tpu_sc as plsc

N_TOTAL = 1000000
RHO = 0.99
ALPHA = 0.5
S_INIT = 2.0
TEMP = 0.5
B = 16384
D = 128

BLK = 2048
G = B // BLK
NC, NS, L = 2, 16, 16
NW = NC * NS
LCH = N_TOTAL // NW
WS = LCH + 6
BP = B // NW
TWO_T2 = 2.0 * TEMP ** 2.0
NPOW2 = 999999995904.0
assert N_TOTAL % NW == 0 and WS % 8 == 0 and B % (L * NW) == 0


def _norm_rows(x):
    n = jnp.sqrt(jnp.sum(x * x, axis=1, keepdims=True))
    return x / jnp.maximum(n, 1e-12)


def _tc_body(a_ref, b_ref, bnd_a_ref, bnd_b_ref, u_ref, qr_ref, att_ref):
    a = _norm_rows(a_ref[...])
    b = _norm_rows(b_ref[...])
    a_bnd = _norm_rows(bnd_a_ref[0])
    b_bnd = _norm_rows(bnd_b_ref[0])
    a_neg = jnp.concatenate([a[1:], a_bnd], axis=0)
    b_neg = jnp.concatenate([b[1:], b_bnd], axis=0)

    def d2(x, y):
        diff = x - y + 1e-6
        return jnp.sum(diff * diff, axis=1)

    d2_attr_a = d2(a, b)
    d2_attr_b = d2(b, a)
    qa = jnp.exp(-d2_attr_a / TWO_T2)
    qb = jnp.exp(-d2_attr_b / TWO_T2)
    qra = jnp.exp(-d2(a, b_neg) / TWO_T2)
    qrb = jnp.exp(-d2(b, a_neg) / TWO_T2)
    u_ref[...] = ((1.0 - RHO) * NPOW2 * 0.25) * (qa + qb + qra + qrb)
    qr_ref[...] = qra + qrb
    att_ref[...] = (d2_attr_a + d2_attr_b) / TWO_T2


def _tc_call(feats_a, feats_b, bnd_a, bnd_b):
    return pl.pallas_call(
        _tc_body,
        grid=(G,),
        in_specs=[
            pl.BlockSpec((BLK, D), lambda j: (j, 0)),
            pl.BlockSpec((BLK, D), lambda j: (j, 0)),
            pl.BlockSpec((1, 1, D), lambda j: (j, 0, 0)),
            pl.BlockSpec((1, 1, D), lambda j: (j, 0, 0)),
        ],
        out_specs=[
            pl.BlockSpec((BLK,), lambda j: (j,)),
            pl.BlockSpec((BLK,), lambda j: (j,)),
            pl.BlockSpec((BLK,), lambda j: (j,)),
        ],
        out_shape=[
            jax.ShapeDtypeStruct((B,), jnp.float32),
            jax.ShapeDtypeStruct((B,), jnp.float32),
            jax.ShapeDtypeStruct((B,), jnp.float32),
        ],
    )(feats_a, feats_b, bnd_a, bnd_b)


def _sc_kernel_body(idx_hbm, u_hbm, qr_hbm, att_hbm, sinv_hbm,
                    out_hbm, part_hbm,
                    idx_v, u_v, qr_v, att_v, tbl_v, acc_v):
    wid = lax.axis_index("s") * NC + lax.axis_index("c")
    lstart = wid * LCH
    wstart = pl.multiple_of(lstart - lax.rem(lstart, 8), 8)
    bbase = wid * BP

    pltpu.sync_copy(idx_hbm, idx_v)
    pltpu.sync_copy(u_hbm, u_v)
    pltpu.sync_copy(qr_hbm, qr_v)
    pltpu.sync_copy(att_hbm.at[pl.ds(bbase, BP)], att_v)
    pltpu.sync_copy(sinv_hbm.at[pl.ds(wstart, WS)], tbl_v)

    npow2 = jnp.float32(NPOW2)
    rho = jnp.float32(RHO)
    zeros = jnp.zeros((L,), jnp.float32)

    def p1(t, acc):
        sl = pl.ds(t * L, L)
        idxl = idx_v[sl]
        local = idxl - wstart
        m = (local >= 0) & (local < WS)
        m_loss = (idxl >= lstart) & (idxl < lstart + LCH)
        lc = jnp.clip(local, 0, WS - 1)
        s_old = plsc.load_gather(tbl_v, [lc], mask=m)
        u_v[sl] = rho * s_old + u_v[sl]
        rep = qr_v[sl] / (s_old / npow2)
        return acc + jnp.where(m_loss, rep, zeros)

    acc = lax.fori_loop(0, B // L, p1, zeros)

    def p_att(t, acc):
        return acc + att_v[pl.ds(t * L, L)]

    acc = lax.fori_loop(0, BP // L, p_att, acc)

    def p2(t, carry):
        sl = pl.ds(t * L, L)
        local = idx_v[sl] - wstart
        m = (local >= 0) & (local < WS)
        lc = jnp.clip(local, 0, WS - 1)
        plsc.store_scatter(tbl_v, [lc], u_v[sl], mask=m)
        return carry

    lax.fori_loop(0, B // L, p2, 0)

    pltpu.sync_copy(tbl_v, out_hbm.at[pl.ds(wstart, WS)])
    acc_v[...] = acc
    pltpu.sync_copy(acc_v, part_hbm.at[wid])


def _sc_call(feats_idx, u, qr, att, s_inv):
    mesh = plsc.VectorSubcoreMesh(
        core_axis_name="c", subcore_axis_name="s",
        num_cores=NC, num_subcores=NS)
    fn = pl.kernel(
        _sc_kernel_body,
        compiler_params=pltpu.CompilerParams(needs_layout_passes=False),
        out_type=[
            jax.ShapeDtypeStruct((N_TOTAL,), jnp.float32),
            jax.ShapeDtypeStruct((NW, L), jnp.float32),
        ],
        mesh=mesh,
        scratch_types=[
            pltpu.VMEM((B,), jnp.int32),
            pltpu.VMEM((B,), jnp.float32),
            pltpu.VMEM((B,), jnp.float32),
            pltpu.VMEM((BP,), jnp.float32),
            pltpu.VMEM((WS,), jnp.float32),
            pltpu.VMEM((L,), jnp.float32),
        ],
    )
    return fn(feats_idx, u, qr, att, s_inv)


def kernel(feats_a, feats_b, feats_idx, s_inv):
    bnd_a = jnp.roll(feats_a[::BLK], -1, axis=0)[:, None, :]
    bnd_b = jnp.roll(feats_b[::BLK], -1, axis=0)[:, None, :]
    u, qr, att = _tc_call(feats_a, feats_b, bnd_a, bnd_b)
    new_s_inv, parts = _sc_call(feats_idx, u, qr, att, s_inv)
    loss = 0.5 * jnp.sum(parts) / B
    return loss, new_s_inv


# speedup vs baseline: 2.5037x; 1.5393x over previous
import functools

import jax
import jax.numpy as jnp
from jax import lax
from jax.experimental import pallas as pl
from jax.experimental.pallas import tpu as pltpu
from jax.experimental.pallas import tpu_sc as plsc

N_TOTAL = 1000000
RHO = 0.99
ALPHA = 0.5
S_INIT = 2.0
TEMP = 0.5
B = 16384
D = 128

BLK = 2048
G = B // BLK
NC, NS, L = 2, 16, 16
NW = NC * NS
LCH = N_TOTAL // NW
WS = LCH + 6
BP = B // NW
TWO_T2 = 2.0 * TEMP ** 2.0
NPOW2 = 999999995904.0
assert N_TOTAL % NW == 0 and WS % 8 == 0 and B % (L * NW) == 0


def _tc_body(a_ref, b_ref, bnd_a_ref, bnd_b_ref, u_ref, qr_ref, att_ref):
    ones_p = jnp.ones((1, D), jnp.float32)

    def rsp(x):
        return lax.dot_general(ones_p, x, (((1,), (1,)), ((), ())),
                               preferred_element_type=jnp.float32,
                               precision=lax.Precision.DEFAULT)

    a = a_ref[...]
    b = b_ref[...]
    last = lax.broadcasted_iota(jnp.int32, (BLK, D), 0) == (BLK - 1)
    a_neg = jnp.where(last, bnd_a_ref[0:1], pltpu.roll(a, BLK - 1, 0))
    b_neg = jnp.where(last, bnd_b_ref[0:1], pltpu.roll(b, BLK - 1, 0))

    na2, nb2 = rsp(a * a), rsp(b * b)
    nan2, nbn2 = rsp(a_neg * a_neg), rsp(b_neg * b_neg)
    sa, sb = rsp(a), rsp(b)
    san, sbn = rsp(a_neg), rsp(b_neg)
    tab, tabn, tban = rsp(a * b), rsp(a * b_neg), rsp(b * a_neg)

    eps = 1e-6
    deps2 = D * eps * eps

    def cn(n2):
        return jnp.maximum(jnp.sqrt(n2), 1e-12)

    ia, ib = 1.0 / cn(na2), 1.0 / cn(nb2)
    ian, ibn = 1.0 / cn(nan2), 1.0 / cn(nbn2)
    q1a, q1b = na2 * ia * ia, nb2 * ib * ib
    q1an, q1bn = nan2 * ian * ian, nbn2 * ibn * ibn
    ea, eb = eps * (sa * ia), eps * (sb * ib)
    ean, ebn = eps * (san * ian), eps * (sbn * ibn)

    d2_attr_a = q1a + q1b + deps2 - 2.0 * (tab * ia * ib) + 2.0 * (ea - eb)
    d2_attr_b = q1a + q1b + deps2 - 2.0 * (tab * ia * ib) + 2.0 * (eb - ea)
    d2_rep_a = q1a + q1bn + deps2 - 2.0 * (tabn * ia * ibn) + 2.0 * (ea - ebn)
    d2_rep_b = q1b + q1an + deps2 - 2.0 * (tban * ib * ian) + 2.0 * (eb - ean)

    qa = jnp.exp(-d2_attr_a / TWO_T2)
    qb = jnp.exp(-d2_attr_b / TWO_T2)
    qra = jnp.exp(-d2_rep_a / TWO_T2)
    qrb = jnp.exp(-d2_rep_b / TWO_T2)
    u_ref[...] = (((1.0 - RHO) * NPOW2 * 0.25)
                  * (qa + qb + qra + qrb)).reshape(BLK)
    qr_ref[...] = (qra + qrb).reshape(BLK)
    att_ref[...] = ((d2_attr_a + d2_attr_b) / TWO_T2).reshape(BLK)


def _tc_call(feats_a, feats_b):
    nxt = lambda j: (((j + 1) % G) * (BLK // 8), 0)
    return pl.pallas_call(
        _tc_body,
        grid=(G,),
        in_specs=[
            pl.BlockSpec((BLK, D), lambda j: (j, 0)),
            pl.BlockSpec((BLK, D), lambda j: (j, 0)),
            pl.BlockSpec((8, D), nxt),
            pl.BlockSpec((8, D), nxt),
        ],
        out_specs=[
            pl.BlockSpec((BLK,), lambda j: (j,)),
            pl.BlockSpec((BLK,), lambda j: (j,)),
            pl.BlockSpec((BLK,), lambda j: (j,)),
        ],
        out_shape=[
            jax.ShapeDtypeStruct((B,), jnp.float32),
            jax.ShapeDtypeStruct((B,), jnp.float32),
            jax.ShapeDtypeStruct((B,), jnp.float32),
        ],
    )(feats_a, feats_b, feats_a, feats_b)


def _sc_kernel_body(idx_hbm, u_hbm, qr_hbm, att_hbm, sinv_hbm,
                    out_hbm, part_hbm,
                    idx_v, u_v, qr_v, att_v, tbl_v, tbl2_v, acc_v, sem):
    wid = lax.axis_index("s") * NC + lax.axis_index("c")
    lstart = wid * LCH
    wstart = pl.multiple_of(lstart - lax.rem(lstart, 8), 8)
    bbase = wid * BP

    win = sinv_hbm.at[pl.ds(wstart, WS)]
    copies = [
        pltpu.async_copy(idx_hbm, idx_v, sem),
        pltpu.async_copy(u_hbm, u_v, sem),
        pltpu.async_copy(qr_hbm, qr_v, sem),
        pltpu.async_copy(att_hbm.at[pl.ds(bbase, BP)], att_v, sem),
        pltpu.async_copy(win, tbl_v, sem),
        pltpu.async_copy(win, tbl2_v, sem),
    ]
    for c in copies:
        c.wait()

    npow2 = jnp.float32(NPOW2)
    rho = jnp.float32(RHO)
    zeros = jnp.zeros((L,), jnp.float32)

    def body(t, acc):
        sl = pl.ds(t * L, L)
        idxl = idx_v[sl]
        local = idxl - wstart
        m = (local >= 0) & (local < WS)
        m_loss = (idxl >= lstart) & (idxl < lstart + LCH)
        lc = jnp.clip(local, 0, WS - 1)
        s_old = plsc.load_gather(tbl_v, [lc], mask=m)
        plsc.store_scatter(tbl2_v, [lc], rho * s_old + u_v[sl], mask=m)
        rep = qr_v[sl] / (s_old / npow2)
        return acc + jnp.where(m_loss, rep, zeros)

    acc = lax.fori_loop(0, B // L, body, zeros)

    def p_att(t, acc):
        return acc + att_v[pl.ds(t * L, L)]

    acc = lax.fori_loop(0, BP // L, p_att, acc)

    pltpu.sync_copy(tbl2_v, out_hbm.at[pl.ds(wstart, WS)])
    acc_v[...] = acc
    pltpu.sync_copy(acc_v, part_hbm.at[wid])


def _sc_call(feats_idx, u, qr, att, s_inv):
    mesh = plsc.VectorSubcoreMesh(
        core_axis_name="c", subcore_axis_name="s",
        num_cores=NC, num_subcores=NS)
    fn = pl.kernel(
        _sc_kernel_body,
        compiler_params=pltpu.CompilerParams(needs_layout_passes=False),
        out_type=[
            jax.ShapeDtypeStruct((N_TOTAL,), jnp.float32),
            jax.ShapeDtypeStruct((NW, L), jnp.float32),
        ],
        mesh=mesh,
        scratch_types=[
            pltpu.VMEM((B,), jnp.int32),
            pltpu.VMEM((B,), jnp.float32),
            pltpu.VMEM((B,), jnp.float32),
            pltpu.VMEM((BP,), jnp.float32),
            pltpu.VMEM((WS,), jnp.float32),
            pltpu.VMEM((WS,), jnp.float32),
            pltpu.VMEM((L,), jnp.float32),
            pltpu.SemaphoreType.DMA,
        ],
    )
    return fn(feats_idx, u, qr, att, s_inv)


def kernel(feats_a, feats_b, feats_idx, s_inv):
    u, qr, att = _tc_call(feats_a, feats_b)
    new_s_inv, parts = _sc_call(feats_idx, u, qr, att, s_inv)
    loss = 0.5 * jnp.sum(parts) / B
    return loss, new_s_inv


# speedup vs baseline: 2.8740x; 1.1479x over previous
import functools

import jax
import jax.numpy as jnp
from jax import lax
from jax.experimental import pallas as pl
from jax.experimental.pallas import tpu as pltpu
from jax.experimental.pallas import tpu_sc as plsc

N_TOTAL = 1000000
RHO = 0.99
ALPHA = 0.5
S_INIT = 2.0
TEMP = 0.5
B = 16384
D = 128

BLK = 2048
G = B // BLK
NC, NS, L = 2, 16, 16
NW = NC * NS
LCH = N_TOTAL // NW
WS = LCH + 6
BP = B // NW
TWO_T2 = 2.0 * TEMP ** 2.0
NPOW2 = 999999995904.0
assert N_TOTAL % NW == 0 and WS % 8 == 0 and B % (L * NW) == 0


def _tc_body(a_ref, b_ref, bnd_a_ref, bnd_b_ref, u_ref, qr_ref, att_ref):
    ones_p = jnp.ones((1, D), jnp.float32)

    def rsp(x):
        return lax.dot_general(ones_p, x, (((1,), (1,)), ((), ())),
                               preferred_element_type=jnp.float32,
                               precision=lax.Precision.DEFAULT)

    a = a_ref[...]
    b = b_ref[...]
    last = lax.broadcasted_iota(jnp.int32, (BLK, D), 0) == (BLK - 1)
    a_neg = jnp.where(last, bnd_a_ref[0:1], pltpu.roll(a, BLK - 1, 0))
    b_neg = jnp.where(last, bnd_b_ref[0:1], pltpu.roll(b, BLK - 1, 0))

    na2, nb2 = rsp(a * a), rsp(b * b)
    nan2, nbn2 = rsp(a_neg * a_neg), rsp(b_neg * b_neg)
    sa, sb = rsp(a), rsp(b)
    san, sbn = rsp(a_neg), rsp(b_neg)
    tab, tabn, tban = rsp(a * b), rsp(a * b_neg), rsp(b * a_neg)

    eps = 1e-6
    deps2 = D * eps * eps

    def cn(n2):
        return jnp.maximum(jnp.sqrt(n2), 1e-12)

    ia, ib = 1.0 / cn(na2), 1.0 / cn(nb2)
    ian, ibn = 1.0 / cn(nan2), 1.0 / cn(nbn2)
    q1a, q1b = na2 * ia * ia, nb2 * ib * ib
    q1an, q1bn = nan2 * ian * ian, nbn2 * ibn * ibn
    ea, eb = eps * (sa * ia), eps * (sb * ib)
    ean, ebn = eps * (san * ian), eps * (sbn * ibn)

    d2_attr_a = q1a + q1b + deps2 - 2.0 * (tab * ia * ib) + 2.0 * (ea - eb)
    d2_attr_b = q1a + q1b + deps2 - 2.0 * (tab * ia * ib) + 2.0 * (eb - ea)
    d2_rep_a = q1a + q1bn + deps2 - 2.0 * (tabn * ia * ibn) + 2.0 * (ea - ebn)
    d2_rep_b = q1b + q1an + deps2 - 2.0 * (tban * ib * ian) + 2.0 * (eb - ean)

    qa = jnp.exp(-d2_attr_a / TWO_T2)
    qb = jnp.exp(-d2_attr_b / TWO_T2)
    qra = jnp.exp(-d2_rep_a / TWO_T2)
    qrb = jnp.exp(-d2_rep_b / TWO_T2)
    u_ref[...] = (((1.0 - RHO) * NPOW2 * 0.25)
                  * (qa + qb + qra + qrb)).reshape(BLK)
    qr_ref[...] = (qra + qrb).reshape(BLK)
    att_ref[...] = ((d2_attr_a + d2_attr_b) / TWO_T2).reshape(BLK)


def _tc_call(feats_a, feats_b):
    nxt = lambda j: (((j + 1) % G) * (BLK // 8), 0)
    return pl.pallas_call(
        _tc_body,
        grid=(G,),
        in_specs=[
            pl.BlockSpec((BLK, D), lambda j: (j, 0)),
            pl.BlockSpec((BLK, D), lambda j: (j, 0)),
            pl.BlockSpec((8, D), nxt),
            pl.BlockSpec((8, D), nxt),
        ],
        out_specs=[
            pl.BlockSpec((BLK,), lambda j: (j,)),
            pl.BlockSpec((BLK,), lambda j: (j,)),
            pl.BlockSpec((BLK,), lambda j: (j,)),
        ],
        out_shape=[
            jax.ShapeDtypeStruct((B,), jnp.float32),
            jax.ShapeDtypeStruct((B,), jnp.float32),
            jax.ShapeDtypeStruct((B,), jnp.float32),
        ],
    )(feats_a, feats_b, feats_a, feats_b)


def _sc_kernel_body(idx_hbm, u_hbm, qr_hbm, att_hbm, sinv_hbm,
                    out_hbm, part_hbm,
                    idx_v, u_v, lc_v, idxs_v, sold_v, qr_v, att_v, tbl_v,
                    acc_v, sem):
    wid = lax.axis_index("s") * NC + lax.axis_index("c")
    lstart = wid * LCH
    wstart = pl.multiple_of(lstart - lax.rem(lstart, 8), 8)
    bbase = wid * BP

    copies = [
        pltpu.async_copy(idx_hbm, idx_v, sem),
        pltpu.async_copy(u_hbm, u_v, sem),
        pltpu.async_copy(idx_hbm.at[pl.ds(bbase, BP)], idxs_v, sem),
        pltpu.async_copy(qr_hbm.at[pl.ds(bbase, BP)], qr_v, sem),
        pltpu.async_copy(att_hbm.at[pl.ds(bbase, BP)], att_v, sem),
        pltpu.async_copy(sinv_hbm.at[pl.ds(wstart, WS)],
                         tbl_v.at[pl.ds(0, WS)], sem),
    ]
    for c in copies:
        c.wait()
    pltpu.async_copy(sinv_hbm.at[idxs_v], sold_v, sem).wait()

    npow2 = jnp.float32(NPOW2)
    rho = jnp.float32(RHO)
    zeros = jnp.zeros((L,), jnp.float32)

    @plsc.parallel_loop(0, B // L, unroll=4)
    def _pass_a(t):
        sl = pl.ds(t * L, L)
        local = idx_v[sl] - wstart
        m = (local >= 0) & (local < WS)
        lc = jnp.where(m, local, WS)
        s_old = plsc.load_gather(tbl_v, [lc])
        u_v[sl] = rho * s_old + u_v[sl]
        lc_v[sl] = lc

    def p_b(t, carry):
        base = t * (L * 4)
        for k in range(4):
            sl = pl.ds(base + k * L, L)
            plsc.store_scatter(tbl_v, [lc_v[sl]], u_v[sl])
        return carry

    lax.fori_loop(0, B // (L * 4), p_b, 0)

    def p_loss(t, acc):
        sl = pl.ds(t * L, L)
        rep = qr_v[sl] / (sold_v[sl] / npow2)
        return acc + rep + att_v[sl]

    acc = lax.fori_loop(0, BP // L, p_loss, zeros)

    pltpu.sync_copy(tbl_v.at[pl.ds(0, WS)], out_hbm.at[pl.ds(wstart, WS)])
    acc_v[...] = acc
    pltpu.sync_copy(acc_v, part_hbm.at[wid])


def _sc_call(feats_idx, u, qr, att, s_inv):
    mesh = plsc.VectorSubcoreMesh(
        core_axis_name="c", subcore_axis_name="s",
        num_cores=NC, num_subcores=NS)
    fn = pl.kernel(
        _sc_kernel_body,
        compiler_params=pltpu.CompilerParams(needs_layout_passes=False),
        out_type=[
            jax.ShapeDtypeStruct((N_TOTAL,), jnp.float32),
            jax.ShapeDtypeStruct((NW, L), jnp.float32),
        ],
        mesh=mesh,
        scratch_types=[
            pltpu.VMEM((B,), jnp.int32),
            pltpu.VMEM((B,), jnp.float32),
            pltpu.VMEM((B,), jnp.int32),
            pltpu.VMEM((BP,), jnp.int32),
            pltpu.VMEM((BP,), jnp.float32),
            pltpu.VMEM((BP,), jnp.float32),
            pltpu.VMEM((BP,), jnp.float32),
            pltpu.VMEM((WS + 8,), jnp.float32),
            pltpu.VMEM((L,), jnp.float32),
            pltpu.SemaphoreType.DMA,
        ],
    )
    return fn(feats_idx, u, qr, att, s_inv)


def kernel(feats_a, feats_b, feats_idx, s_inv):
    u, qr, att = _tc_call(feats_a, feats_b)
    new_s_inv, parts = _sc_call(feats_idx, u, qr, att, s_inv)
    loss = 0.5 * jnp.sum(parts) / B
    return loss, new_s_inv


# speedup vs baseline: 2.9721x; 1.0341x over previous
import functools

import jax
import jax.numpy as jnp
from jax import lax
from jax.experimental import pallas as pl
from jax.experimental.pallas import tpu as pltpu
from jax.experimental.pallas import tpu_sc as plsc

N_TOTAL = 1000000
RHO = 0.99
ALPHA = 0.5
S_INIT = 2.0
TEMP = 0.5
B = 16384
D = 128

BLK = 4096
G = B // BLK
NC, NS, L = 2, 16, 16
NW = NC * NS
LCH = N_TOTAL // NW
WS = LCH + 6
BP = B // NW
TWO_T2 = 2.0 * TEMP ** 2.0
NPOW2 = 999999995904.0
assert N_TOTAL % NW == 0 and WS % 8 == 0 and B % (L * NW) == 0


def _tc_body(a_ref, b_ref, bnd_a_ref, bnd_b_ref, u_ref, qr_ref, att_ref):
    ones_p = jnp.ones((1, D), jnp.float32)

    def rsp(x):
        return lax.dot_general(ones_p, x, (((1,), (1,)), ((), ())),
                               preferred_element_type=jnp.float32,
                               precision=lax.Precision.DEFAULT)

    a = a_ref[...]
    b = b_ref[...]
    last = lax.broadcasted_iota(jnp.int32, (BLK, D), 0) == (BLK - 1)
    a_neg = jnp.where(last, bnd_a_ref[0:1], pltpu.roll(a, BLK - 1, 0))
    b_neg = jnp.where(last, bnd_b_ref[0:1], pltpu.roll(b, BLK - 1, 0))

    na2, nb2 = rsp(a * a), rsp(b * b)
    nan2, nbn2 = rsp(a_neg * a_neg), rsp(b_neg * b_neg)
    sa, sb = rsp(a), rsp(b)
    san, sbn = rsp(a_neg), rsp(b_neg)
    tab, tabn, tban = rsp(a * b), rsp(a * b_neg), rsp(b * a_neg)

    eps = 1e-6
    deps2 = D * eps * eps

    def cn(n2):
        return jnp.maximum(jnp.sqrt(n2), 1e-12)

    ia, ib = 1.0 / cn(na2), 1.0 / cn(nb2)
    ian, ibn = 1.0 / cn(nan2), 1.0 / cn(nbn2)
    q1a, q1b = na2 * ia * ia, nb2 * ib * ib
    q1an, q1bn = nan2 * ian * ian, nbn2 * ibn * ibn
    ea, eb = eps * (sa * ia), eps * (sb * ib)
    ean, ebn = eps * (san * ian), eps * (sbn * ibn)

    d2_attr_a = q1a + q1b + deps2 - 2.0 * (tab * ia * ib) + 2.0 * (ea - eb)
    d2_attr_b = q1a + q1b + deps2 - 2.0 * (tab * ia * ib) + 2.0 * (eb - ea)
    d2_rep_a = q1a + q1bn + deps2 - 2.0 * (tabn * ia * ibn) + 2.0 * (ea - ebn)
    d2_rep_b = q1b + q1an + deps2 - 2.0 * (tban * ib * ian) + 2.0 * (eb - ean)

    qa = jnp.exp(-d2_attr_a / TWO_T2)
    qb = jnp.exp(-d2_attr_b / TWO_T2)
    qra = jnp.exp(-d2_rep_a / TWO_T2)
    qrb = jnp.exp(-d2_rep_b / TWO_T2)
    u_ref[...] = (((1.0 - RHO) * NPOW2 * 0.25)
                  * (qa + qb + qra + qrb)).reshape(BLK)
    qr_ref[...] = (qra + qrb).reshape(BLK)
    att_ref[...] = ((d2_attr_a + d2_attr_b) / TWO_T2).reshape(BLK)


def _tc_call(feats_a, feats_b):
    nxt = lambda j: (((j + 1) % G) * (BLK // 8), 0)
    return pl.pallas_call(
        _tc_body,
        grid=(G,),
        in_specs=[
            pl.BlockSpec((BLK, D), lambda j: (j, 0)),
            pl.BlockSpec((BLK, D), lambda j: (j, 0)),
            pl.BlockSpec((8, D), nxt),
            pl.BlockSpec((8, D), nxt),
        ],
        out_specs=[
            pl.BlockSpec((BLK,), lambda j: (j,)),
            pl.BlockSpec((BLK,), lambda j: (j,)),
            pl.BlockSpec((BLK,), lambda j: (j,)),
        ],
        out_shape=[
            jax.ShapeDtypeStruct((B,), jnp.float32),
            jax.ShapeDtypeStruct((B,), jnp.float32),
            jax.ShapeDtypeStruct((B,), jnp.float32),
        ],
    )(feats_a, feats_b, feats_a, feats_b)


def _sc_kernel_body(idx_hbm, u_hbm, qr_hbm, att_hbm, sinv_hbm,
                    out_hbm, part_hbm,
                    idx_v, u_v, lc_v, idxs_v, sold_v, qr_v, att_v, tbl_v,
                    acc_v, sem, sem2):
    wid = lax.axis_index("s") * NC + lax.axis_index("c")
    lstart = wid * LCH
    wstart = pl.multiple_of(lstart - lax.rem(lstart, 8), 8)
    bbase = wid * BP

    copies = [
        pltpu.async_copy(idx_hbm, idx_v, sem),
        pltpu.async_copy(u_hbm, u_v, sem),
        pltpu.async_copy(idx_hbm.at[pl.ds(bbase, BP)], idxs_v, sem),
        pltpu.async_copy(qr_hbm.at[pl.ds(bbase, BP)], qr_v, sem),
        pltpu.async_copy(att_hbm.at[pl.ds(bbase, BP)], att_v, sem),
        pltpu.async_copy(sinv_hbm.at[pl.ds(wstart, WS)],
                         tbl_v.at[pl.ds(0, WS)], sem),
    ]
    for c in copies:
        c.wait()
    rep_h = pltpu.async_copy(sinv_hbm.at[idxs_v], sold_v, sem2)

    npow2 = jnp.float32(NPOW2)
    rho = jnp.float32(RHO)
    zeros = jnp.zeros((L,), jnp.float32)

    @plsc.parallel_loop(0, B // L, unroll=4)
    def _pass_a(t):
        sl = pl.ds(t * L, L)
        local = idx_v[sl] - wstart
        m = (local >= 0) & (local < WS)
        lc = jnp.where(m, local, WS)
        s_old = plsc.load_gather(tbl_v, [lc])
        u_v[sl] = rho * s_old + u_v[sl]
        lc_v[sl] = lc

    def p_b(t, carry):
        base = t * (L * 4)
        for k in range(4):
            sl = pl.ds(base + k * L, L)
            plsc.store_scatter(tbl_v, [lc_v[sl]], u_v[sl])
        return carry

    lax.fori_loop(0, B // (L * 4), p_b, 0)

    wb_h = pltpu.async_copy(tbl_v.at[pl.ds(0, WS)],
                            out_hbm.at[pl.ds(wstart, WS)], sem)

    rep_h.wait()

    def p_loss(t, acc):
        sl = pl.ds(t * L, L)
        rep = qr_v[sl] / (sold_v[sl] / npow2)
        return acc + rep + att_v[sl]

    acc = lax.fori_loop(0, BP // L, p_loss, zeros)
    acc_v[...] = acc
    pltpu.sync_copy(acc_v, part_hbm.at[wid])
    wb_h.wait()


def _sc_call(feats_idx, u, qr, att, s_inv):
    mesh = plsc.VectorSubcoreMesh(
        core_axis_name="c", subcore_axis_name="s",
        num_cores=NC, num_subcores=NS)
    fn = pl.kernel(
        _sc_kernel_body,
        compiler_params=pltpu.CompilerParams(needs_layout_passes=False),
        out_type=[
            jax.ShapeDtypeStruct((N_TOTAL,), jnp.float32),
            jax.ShapeDtypeStruct((NW, L), jnp.float32),
        ],
        mesh=mesh,
        scratch_types=[
            pltpu.VMEM((B,), jnp.int32),
            pltpu.VMEM((B,), jnp.float32),
            pltpu.VMEM((B,), jnp.int32),
            pltpu.VMEM((BP,), jnp.int32),
            pltpu.VMEM((BP,), jnp.float32),
            pltpu.VMEM((BP,), jnp.float32),
            pltpu.VMEM((BP,), jnp.float32),
            pltpu.VMEM((WS + 8,), jnp.float32),
            pltpu.VMEM((L,), jnp.float32),
            pltpu.SemaphoreType.DMA,
            pltpu.SemaphoreType.DMA,
        ],
    )
    return fn(feats_idx, u, qr, att, s_inv)


def kernel(feats_a, feats_b, feats_idx, s_inv):
    u, qr, att = _tc_call(feats_a, feats_b)
    new_s_inv, parts = _sc_call(feats_idx, u, qr, att, s_inv)
    loss = 0.5 * jnp.sum(parts) / B
    return loss, new_s_inv


# speedup vs baseline: 2.9723x; 1.0001x over previous
import functools

import jax
import jax.numpy as jnp
from jax import lax
from jax.experimental import pallas as pl
from jax.experimental.pallas import tpu as pltpu
from jax.experimental.pallas import tpu_sc as plsc

N_TOTAL = 1000000
RHO = 0.99
ALPHA = 0.5
S_INIT = 2.0
TEMP = 0.5
B = 16384
D = 128

BLK = 4096
G = B // BLK
NC, NS, L = 2, 16, 16
NW = NC * NS
LCH = N_TOTAL // NW
WS = LCH + 6
BP = B // NW
TWO_T2 = 2.0 * TEMP ** 2.0
NPOW2 = 999999995904.0
assert N_TOTAL % NW == 0 and WS % 8 == 0 and B % (L * NW) == 0


def _tc_body(a_ref, b_ref, bnd_a_ref, bnd_b_ref, u_ref, qr_ref, att_ref):
    ones_p = jnp.ones((1, D), jnp.float32)

    def rsp(x):
        return lax.dot_general(ones_p, x, (((1,), (1,)), ((), ())),
                               preferred_element_type=jnp.float32,
                               precision=lax.Precision.DEFAULT)

    a = a_ref[...]
    b = b_ref[...]
    last = lax.broadcasted_iota(jnp.int32, (BLK, D), 0) == (BLK - 1)
    a_neg = jnp.where(last, bnd_a_ref[0:1], pltpu.roll(a, BLK - 1, 0))
    b_neg = jnp.where(last, bnd_b_ref[0:1], pltpu.roll(b, BLK - 1, 0))

    na2, nb2 = rsp(a * a), rsp(b * b)
    nan2, nbn2 = rsp(a_neg * a_neg), rsp(b_neg * b_neg)
    sa, sb = rsp(a), rsp(b)
    san, sbn = rsp(a_neg), rsp(b_neg)
    tab, tabn, tban = rsp(a * b), rsp(a * b_neg), rsp(b * a_neg)

    eps = 1e-6
    deps2 = D * eps * eps

    def cn(n2):
        return jnp.maximum(jnp.sqrt(n2), 1e-12)

    ia, ib = 1.0 / cn(na2), 1.0 / cn(nb2)
    ian, ibn = 1.0 / cn(nan2), 1.0 / cn(nbn2)
    q1a, q1b = na2 * ia * ia, nb2 * ib * ib
    q1an, q1bn = nan2 * ian * ian, nbn2 * ibn * ibn
    ea, eb = eps * (sa * ia), eps * (sb * ib)
    ean, ebn = eps * (san * ian), eps * (sbn * ibn)

    d2_attr_a = q1a + q1b + deps2 - 2.0 * (tab * ia * ib) + 2.0 * (ea - eb)
    d2_attr_b = q1a + q1b + deps2 - 2.0 * (tab * ia * ib) + 2.0 * (eb - ea)
    d2_rep_a = q1a + q1bn + deps2 - 2.0 * (tabn * ia * ibn) + 2.0 * (ea - ebn)
    d2_rep_b = q1b + q1an + deps2 - 2.0 * (tban * ib * ian) + 2.0 * (eb - ean)

    qa = jnp.exp(-d2_attr_a / TWO_T2)
    qb = jnp.exp(-d2_attr_b / TWO_T2)
    qra = jnp.exp(-d2_rep_a / TWO_T2)
    qrb = jnp.exp(-d2_rep_b / TWO_T2)
    u_ref[...] = (((1.0 - RHO) * NPOW2 * 0.25)
                  * (qa + qb + qra + qrb)).reshape(BLK)
    qr_ref[...] = (qra + qrb).reshape(BLK)
    att_ref[...] = ((d2_attr_a + d2_attr_b) / TWO_T2).reshape(BLK)


def _tc_call(feats_a, feats_b):
    nxt = lambda j: (((j + 1) % G) * (BLK // 8), 0)
    return pl.pallas_call(
        _tc_body,
        grid=(G,),
        in_specs=[
            pl.BlockSpec((BLK, D), lambda j: (j, 0)),
            pl.BlockSpec((BLK, D), lambda j: (j, 0)),
            pl.BlockSpec((8, D), nxt),
            pl.BlockSpec((8, D), nxt),
        ],
        out_specs=[
            pl.BlockSpec((BLK,), lambda j: (j,)),
            pl.BlockSpec((BLK,), lambda j: (j,)),
            pl.BlockSpec((BLK,), lambda j: (j,)),
        ],
        out_shape=[
            jax.ShapeDtypeStruct((B,), jnp.float32),
            jax.ShapeDtypeStruct((B,), jnp.float32),
            jax.ShapeDtypeStruct((B,), jnp.float32),
        ],
    )(feats_a, feats_b, feats_a, feats_b)


def _sc_kernel_body(idx_hbm, u_hbm, qr_hbm, att_hbm, sinv_hbm,
                    out_hbm, part_hbm,
                    idx_v, u_v, lc_v, idxs_v, sold_v, qr_v, att_v, tbl_v,
                    acc_v, sem, sem2):
    wid = lax.axis_index("s") * NC + lax.axis_index("c")
    lstart = wid * LCH
    wstart = pl.multiple_of(lstart - lax.rem(lstart, 8), 8)
    bbase = wid * BP

    copies = [
        pltpu.async_copy(idx_hbm, idx_v, sem),
        pltpu.async_copy(u_hbm, u_v, sem),
        pltpu.async_copy(idx_hbm.at[pl.ds(bbase, BP)], idxs_v, sem),
        pltpu.async_copy(qr_hbm.at[pl.ds(bbase, BP)], qr_v, sem),
        pltpu.async_copy(att_hbm.at[pl.ds(bbase, BP)], att_v, sem),
        pltpu.async_copy(sinv_hbm.at[pl.ds(wstart, WS)],
                         tbl_v.at[pl.ds(0, WS)], sem),
    ]
    for c in copies:
        c.wait()
    rep_h = pltpu.async_copy(sinv_hbm.at[idxs_v], sold_v, sem2)

    npow2 = jnp.float32(NPOW2)
    rho = jnp.float32(RHO)
    zeros = jnp.zeros((L,), jnp.float32)

    @plsc.parallel_loop(0, B // L, unroll=8)
    def _pass_a(t):
        sl = pl.ds(t * L, L)
        local = idx_v[sl] - wstart
        m = (local >= 0) & (local < WS)
        lc = jnp.where(m, local, WS)
        s_old = plsc.load_gather(tbl_v, [lc])
        u_v[sl] = rho * s_old + u_v[sl]
        lc_v[sl] = lc

    def p_b(t, carry):
        base = t * (L * 8)
        for k in range(8):
            sl = pl.ds(base + k * L, L)
            plsc.store_scatter(tbl_v, [lc_v[sl]], u_v[sl])
        return carry

    lax.fori_loop(0, B // (L * 8), p_b, 0)

    wb_h = pltpu.async_copy(tbl_v.at[pl.ds(0, WS)],
                            out_hbm.at[pl.ds(wstart, WS)], sem)

    rep_h.wait()

    def p_loss(t, acc):
        sl = pl.ds(t * L, L)
        rep = qr_v[sl] / (sold_v[sl] / npow2)
        return acc + rep + att_v[sl]

    acc = lax.fori_loop(0, BP // L, p_loss, zeros)
    acc_v[...] = acc
    pltpu.sync_copy(acc_v, part_hbm.at[wid])
    wb_h.wait()


def _sc_call(feats_idx, u, qr, att, s_inv):
    mesh = plsc.VectorSubcoreMesh(
        core_axis_name="c", subcore_axis_name="s",
        num_cores=NC, num_subcores=NS)
    fn = pl.kernel(
        _sc_kernel_body,
        compiler_params=pltpu.CompilerParams(needs_layout_passes=False),
        out_type=[
            jax.ShapeDtypeStruct((N_TOTAL,), jnp.float32),
            jax.ShapeDtypeStruct((NW, L), jnp.float32),
        ],
        mesh=mesh,
        scratch_types=[
            pltpu.VMEM((B,), jnp.int32),
            pltpu.VMEM((B,), jnp.float32),
            pltpu.VMEM((B,), jnp.int32),
            pltpu.VMEM((BP,), jnp.int32),
            pltpu.VMEM((BP,), jnp.float32),
            pltpu.VMEM((BP,), jnp.float32),
            pltpu.VMEM((BP,), jnp.float32),
            pltpu.VMEM((WS + 8,), jnp.float32),
            pltpu.VMEM((L,), jnp.float32),
            pltpu.SemaphoreType.DMA,
            pltpu.SemaphoreType.DMA,
        ],
    )
    return fn(feats_idx, u, qr, att, s_inv)


def kernel(feats_a, feats_b, feats_idx, s_inv):
    u, qr, att = _tc_call(feats_a, feats_b)
    new_s_inv, parts = _sc_call(feats_idx, u, qr, att, s_inv)
    loss = 0.5 * jnp.sum(parts) / B
    return loss, new_s_inv


# speedup vs baseline: 2.9977x; 1.0085x over previous
import functools

import jax
import jax.numpy as jnp
from jax import lax
from jax.experimental import pallas as pl
from jax.experimental.pallas import tpu as pltpu
from jax.experimental.pallas import tpu_sc as plsc

N_TOTAL = 1000000
RHO = 0.99
ALPHA = 0.5
S_INIT = 2.0
TEMP = 0.5
B = 16384
D = 128

BLK = 4096
G = B // BLK
NC, NS, L = 2, 16, 16
NW = NC * NS
LCH = N_TOTAL // NW
WS = LCH + 6
BP = B // NW
TWO_T2 = 2.0 * TEMP ** 2.0
NPOW2 = 999999995904.0
assert N_TOTAL % NW == 0 and WS % 8 == 0 and B % (L * NW) == 0


def _tc_body(a_ref, b_ref, bnd_a_ref, bnd_b_ref, u_ref, qr_ref, att_ref):
    ones_p = jnp.ones((1, D), jnp.float32)

    def rsp(x):
        return lax.dot_general(ones_p, x, (((1,), (1,)), ((), ())),
                               preferred_element_type=jnp.float32,
                               precision=lax.Precision.DEFAULT)

    a = a_ref[...]
    b = b_ref[...]
    last = lax.broadcasted_iota(jnp.int32, (BLK, D), 0) == (BLK - 1)
    a_neg = jnp.where(last, bnd_a_ref[0:1], pltpu.roll(a, BLK - 1, 0))
    b_neg = jnp.where(last, bnd_b_ref[0:1], pltpu.roll(b, BLK - 1, 0))

    na2, nb2 = rsp(a * a), rsp(b * b)
    nan2, nbn2 = rsp(a_neg * a_neg), rsp(b_neg * b_neg)
    sa, sb = rsp(a), rsp(b)
    san, sbn = rsp(a_neg), rsp(b_neg)
    tab, tabn, tban = rsp(a * b), rsp(a * b_neg), rsp(b * a_neg)

    eps = 1e-6
    deps2 = D * eps * eps

    def cn(n2):
        return jnp.maximum(jnp.sqrt(n2), 1e-12)

    ia, ib = 1.0 / cn(na2), 1.0 / cn(nb2)
    ian, ibn = 1.0 / cn(nan2), 1.0 / cn(nbn2)
    q1a, q1b = na2 * ia * ia, nb2 * ib * ib
    q1an, q1bn = nan2 * ian * ian, nbn2 * ibn * ibn
    ea, eb = eps * (sa * ia), eps * (sb * ib)
    ean, ebn = eps * (san * ian), eps * (sbn * ibn)

    d2_attr_a = q1a + q1b + deps2 - 2.0 * (tab * ia * ib) + 2.0 * (ea - eb)
    d2_attr_b = q1a + q1b + deps2 - 2.0 * (tab * ia * ib) + 2.0 * (eb - ea)
    d2_rep_a = q1a + q1bn + deps2 - 2.0 * (tabn * ia * ibn) + 2.0 * (ea - ebn)
    d2_rep_b = q1b + q1an + deps2 - 2.0 * (tban * ib * ian) + 2.0 * (eb - ean)

    qa = jnp.exp(-d2_attr_a / TWO_T2)
    qb = jnp.exp(-d2_attr_b / TWO_T2)
    qra = jnp.exp(-d2_rep_a / TWO_T2)
    qrb = jnp.exp(-d2_rep_b / TWO_T2)
    u_ref[...] = (((1.0 - RHO) * NPOW2 * 0.25)
                  * (qa + qb + qra + qrb)).reshape(BLK)
    qr_ref[...] = (qra + qrb).reshape(BLK)
    att_ref[...] = ((d2_attr_a + d2_attr_b) / TWO_T2).reshape(BLK)


def _tc_call(feats_a, feats_b):
    nxt = lambda j: (((j + 1) % G) * (BLK // 8), 0)
    return pl.pallas_call(
        _tc_body,
        grid=(G,),
        in_specs=[
            pl.BlockSpec((BLK, D), lambda j: (j, 0)),
            pl.BlockSpec((BLK, D), lambda j: (j, 0)),
            pl.BlockSpec((8, D), nxt),
            pl.BlockSpec((8, D), nxt),
        ],
        out_specs=[
            pl.BlockSpec((BLK,), lambda j: (j,)),
            pl.BlockSpec((BLK,), lambda j: (j,)),
            pl.BlockSpec((BLK,), lambda j: (j,)),
        ],
        out_shape=[
            jax.ShapeDtypeStruct((B,), jnp.float32),
            jax.ShapeDtypeStruct((B,), jnp.float32),
            jax.ShapeDtypeStruct((B,), jnp.float32),
        ],
    )(feats_a, feats_b, feats_a, feats_b)


def _sc_kernel_body(idx_hbm, u_hbm, qr_hbm, att_hbm, sinv_hbm,
                    out_hbm, part_hbm,
                    idx_v, u_v, lc_v, idxs_v, sold_v, qr_v, att_v, tbl_v,
                    acc_v, sem, sem2, semt, semc):
    wid = lax.axis_index("s") * NC + lax.axis_index("c")
    lstart = wid * LCH
    wstart = pl.multiple_of(lstart - lax.rem(lstart, 8), 8)
    bbase = wid * BP

    nchk = 4
    cb = B // nchk
    tbl_h = pltpu.async_copy(sinv_hbm.at[pl.ds(wstart, WS)],
                             tbl_v.at[pl.ds(0, WS)], semt)
    chunk_h = []
    for k in range(nchk):
        s = pl.ds(k * cb, cb)
        chunk_h.append((
            pltpu.async_copy(idx_hbm.at[s], idx_v.at[s], semc.at[k]),
            pltpu.async_copy(u_hbm.at[s], u_v.at[s], semc.at[k])))
    slice_h = [
        pltpu.async_copy(idx_hbm.at[pl.ds(bbase, BP)], idxs_v, sem),
        pltpu.async_copy(qr_hbm.at[pl.ds(bbase, BP)], qr_v, sem),
        pltpu.async_copy(att_hbm.at[pl.ds(bbase, BP)], att_v, sem),
    ]
    tbl_h.wait()
    for c in slice_h:
        c.wait()
    rep_h = pltpu.async_copy(sinv_hbm.at[idxs_v], sold_v, sem2)

    npow2 = jnp.float32(NPOW2)
    rho = jnp.float32(RHO)
    zeros = jnp.zeros((L,), jnp.float32)

    for k in range(nchk):
        for c in chunk_h[k]:
            c.wait()

        @plsc.parallel_loop(k * (cb // L), (k + 1) * (cb // L), unroll=8)
        def _pass_a(t):
            sl = pl.ds(t * L, L)
            local = idx_v[sl] - wstart
            m = (local >= 0) & (local < WS)
            lc = jnp.where(m, local, WS)
            s_old = plsc.load_gather(tbl_v, [lc])
            u_v[sl] = rho * s_old + u_v[sl]
            lc_v[sl] = lc

    def p_b(t, carry):
        base = t * (L * 8)
        for k in range(8):
            sl = pl.ds(base + k * L, L)
            plsc.store_scatter(tbl_v, [lc_v[sl]], u_v[sl])
        return carry

    lax.fori_loop(0, B // (L * 8), p_b, 0)

    wb_h = pltpu.async_copy(tbl_v.at[pl.ds(0, WS)],
                            out_hbm.at[pl.ds(wstart, WS)], sem)

    rep_h.wait()

    def p_loss(t, acc):
        sl = pl.ds(t * L, L)
        rep = qr_v[sl] / (sold_v[sl] / npow2)
        return acc + rep + att_v[sl]

    acc = lax.fori_loop(0, BP // L, p_loss, zeros)
    acc_v[...] = acc
    pltpu.sync_copy(acc_v, part_hbm.at[wid])
    wb_h.wait()


def _sc_call(feats_idx, u, qr, att, s_inv):
    mesh = plsc.VectorSubcoreMesh(
        core_axis_name="c", subcore_axis_name="s",
        num_cores=NC, num_subcores=NS)
    fn = pl.kernel(
        _sc_kernel_body,
        compiler_params=pltpu.CompilerParams(needs_layout_passes=False),
        out_type=[
            jax.ShapeDtypeStruct((N_TOTAL,), jnp.float32),
            jax.ShapeDtypeStruct((NW, L), jnp.float32),
        ],
        mesh=mesh,
        scratch_types=[
            pltpu.VMEM((B,), jnp.int32),
            pltpu.VMEM((B,), jnp.float32),
            pltpu.VMEM((B,), jnp.int32),
            pltpu.VMEM((BP,), jnp.int32),
            pltpu.VMEM((BP,), jnp.float32),
            pltpu.VMEM((BP,), jnp.float32),
            pltpu.VMEM((BP,), jnp.float32),
            pltpu.VMEM((WS + 8,), jnp.float32),
            pltpu.VMEM((L,), jnp.float32),
            pltpu.SemaphoreType.DMA,
            pltpu.SemaphoreType.DMA,
            pltpu.SemaphoreType.DMA,
            pltpu.SemaphoreType.DMA((4,)),
        ],
    )
    return fn(feats_idx, u, qr, att, s_inv)


def kernel(feats_a, feats_b, feats_idx, s_inv):
    u, qr, att = _tc_call(feats_a, feats_b)
    new_s_inv, parts = _sc_call(feats_idx, u, qr, att, s_inv)
    loss = 0.5 * jnp.sum(parts) / B
    return loss, new_s_inv
